# baseline (speedup 1.0000x reference)
import jax
import jax.numpy as jnp
from jax import lax
from jax.experimental import pallas as pl

S = 197
K = 12
NEG_INF = float("-inf")


def _bf16_dot(a, b, dims):
    return lax.dot_general(
        a.astype(jnp.bfloat16), b.astype(jnp.bfloat16),
        (dims, ((), ())), preferred_element_type=jnp.float32)


def _body(x0_ref, x1_ref, x2_ref, r3_ref, out_ref):
    A0 = x0_ref[0, 0, 0]
    A1 = x1_ref[0, 0, 0]
    A2 = x2_ref[0, 0, 0]
    u1 = r3_ref[0, 0, 0, 0:1]

    M1 = _bf16_dot(A1, A0, ((1,), (1,)))
    M2 = _bf16_dot(A2, M1, ((0,), (0,)))
    u4 = _bf16_dot(u1, M2, ((1,), (0,)))

    idxs = lax.broadcasted_iota(jnp.int32, (1, S), 1)
    s = jnp.where(idxs == 0, NEG_INF, u4)
    kiota = lax.broadcasted_iota(jnp.int32, (1, K), 1)
    out = jnp.zeros((1, K), jnp.int32)
    for k in range(K):
        m = jnp.max(s)
        idx = jnp.min(jnp.where(s == m, idxs, S))
        out = jnp.where(kiota == k, idx - 1, out)
        s = jnp.where(idxs == idx, NEG_INF, s)
    out_ref[0, 0] = out


@jax.jit
def kernel(x):
    L, B, H, s1, s2 = x.shape
    assert (s1, s2) == (S, S)

    grid = (B, H)
    mat_spec = lambda l: pl.BlockSpec(
        (1, 1, 1, S, S), lambda b, h, l=l: (l, b, h, 0, 0))
    row_spec = pl.BlockSpec((1, 1, 1, 8, S), lambda b, h: (3, b, h, 0, 0))

    out = pl.pallas_call(
        _body,
        grid=grid,
        in_specs=[mat_spec(0), mat_spec(1), mat_spec(2), row_spec],
        out_specs=pl.BlockSpec((1, 1, 1, K), lambda b, h: (b, h, 0, 0)),
        out_shape=jax.ShapeDtypeStruct((B, H, 1, K), jnp.int32),
    )(x, x, x, x)
    return out.reshape(B, H, K)


# speedup vs baseline: 1.0119x; 1.0119x over previous
import jax
import jax.numpy as jnp
from jax import lax
from jax.experimental import pallas as pl

S = 197
K = 12
NEG_INF = float("-inf")


def _bf16_dot(a, b, dims):
    return lax.dot_general(
        a.astype(jnp.bfloat16), b.astype(jnp.bfloat16),
        (dims, ((), ())), preferred_element_type=jnp.float32)


def _body(x0_ref, x1_ref, x2_ref, r3_ref, out_ref):
    A0 = x0_ref[0, 0, 0]
    A1 = x1_ref[0, 0, 0]
    A2 = x2_ref[0, 0, 0]
    u1 = r3_ref[0, 0, 0, 0:1]

    P = _bf16_dot(A0, A1, ((1,), (1,)))
    Q = _bf16_dot(P, A2, ((1,), (0,)))
    u4 = _bf16_dot(u1, Q, ((1,), (1,)))

    idxs = lax.broadcasted_iota(jnp.int32, (1, S), 1)
    s = jnp.where(idxs == 0, NEG_INF, u4)
    kiota = lax.broadcasted_iota(jnp.int32, (1, K), 1)
    out = jnp.zeros((1, K), jnp.int32)
    for k in range(K):
        m = jnp.max(s)
        idx = jnp.min(jnp.where(s == m, idxs, S))
        out = jnp.where(kiota == k, idx - 1, out)
        s = jnp.where(idxs == idx, NEG_INF, s)
    out_ref[0, 0] = out


@jax.jit
def kernel(x):
    L, B, H, s1, s2 = x.shape
    assert (s1, s2) == (S, S)

    grid = (B, H)
    mat_spec = lambda l: pl.BlockSpec(
        (1, 1, 1, S, S), lambda b, h, l=l: (l, b, h, 0, 0))
    row_spec = pl.BlockSpec((1, 1, 1, 8, S), lambda b, h: (3, b, h, 0, 0))

    out = pl.pallas_call(
        _body,
        grid=grid,
        in_specs=[mat_spec(0), mat_spec(1), mat_spec(2), row_spec],
        out_specs=pl.BlockSpec((1, 1, 1, K), lambda b, h: (b, h, 0, 0)),
        out_shape=jax.ShapeDtypeStruct((B, H, 1, K), jnp.int32),
    )(x, x, x, x)
    return out.reshape(B, H, K)


# speedup vs baseline: 3.1474x; 3.1104x over previous
import jax
import jax.numpy as jnp
from jax import lax
from jax.experimental import pallas as pl

S = 197
K = 12
NEG_INF = float("-inf")


def _bf16_dot(a, b, dims):
    return lax.dot_general(
        a.astype(jnp.bfloat16), b.astype(jnp.bfloat16),
        (dims, ((), ())), preferred_element_type=jnp.float32)


def _body(x0_ref, x1_ref, x2_ref, r3_ref, out_ref):
    A0 = x0_ref[0, 0, 0]
    A1 = x1_ref[0, 0, 0]
    A2 = x2_ref[0, 0, 0]
    u1 = r3_ref[0, 0, 0, 0:1]

    P = _bf16_dot(A0, A1, ((1,), (1,)))
    Q = _bf16_dot(P, A2, ((1,), (0,)))
    u4 = _bf16_dot(u1, Q, ((1,), (1,)))
    u4t = _bf16_dot(Q, r3_ref[0, 0, 0], ((1,), (1,)))[:, 0:1]

    col_i = lax.broadcasted_iota(jnp.int32, (S, 1), 0)
    row_j = lax.broadcasted_iota(jnp.int32, (1, S), 1)
    s_row = jnp.where(row_j == 0, NEG_INF, u4)
    s_col = jnp.where(col_i == 0, NEG_INF, u4t)
    one = jnp.float32(1.0)
    zero = jnp.float32(0.0)
    g = jnp.where(s_row > s_col, one, zero)
    e = jnp.where(s_row == s_col, one, zero)
    lo = jnp.where(row_j < col_i, one, zero)
    C = (g + e * lo).astype(jnp.bfloat16)
    rank = lax.dot_general(C, jnp.ones((S, 1), jnp.bfloat16),
                           (((1,), (0,)), ((), ())),
                           preferred_element_type=jnp.float32)
    kio = lax.broadcasted_iota(jnp.int32, (1, K), 1).astype(jnp.float32)
    onehot = jnp.where(rank == kio, one, zero).astype(jnp.bfloat16)
    iv = (row_j - 1).astype(jnp.float32).astype(jnp.bfloat16)
    out_f = lax.dot_general(iv, onehot, (((1,), (0,)), ((), ())),
                            preferred_element_type=jnp.float32)
    out_ref[0, 0] = out_f.astype(jnp.int32)


@jax.jit
def kernel(x):
    L, B, H, s1, s2 = x.shape
    assert (s1, s2) == (S, S)

    grid = (B, H)
    mat_spec = lambda l: pl.BlockSpec(
        (1, 1, 1, S, S), lambda b, h, l=l: (l, b, h, 0, 0))
    row_spec = pl.BlockSpec((1, 1, 1, 8, S), lambda b, h: (3, b, h, 0, 0))

    out = pl.pallas_call(
        _body,
        grid=grid,
        in_specs=[mat_spec(0), mat_spec(1), mat_spec(2), row_spec],
        out_specs=pl.BlockSpec((1, 1, 1, K), lambda b, h: (b, h, 0, 0)),
        out_shape=jax.ShapeDtypeStruct((B, H, 1, K), jnp.int32),
    )(x, x, x, x)
    return out.reshape(B, H, K)


# speedup vs baseline: 4.0042x; 1.2722x over previous
import jax
import jax.numpy as jnp
from jax import lax
from jax.experimental import pallas as pl

S = 197
K = 12
NEG_INF = float("-inf")


def _bf16_dot(a, b, dims):
    return lax.dot_general(
        a.astype(jnp.bfloat16), b.astype(jnp.bfloat16),
        (dims, ((), ())), preferred_element_type=jnp.float32)


def _one_head(x0_ref, x1_ref, x2_ref, r3_ref, out_ref, h):
    A0 = x0_ref[0, 0, h]
    A1 = x1_ref[0, 0, h]
    A2 = x2_ref[0, 0, h]
    u1 = r3_ref[0, 0, h, 0:1]

    P = _bf16_dot(A0, A1, ((1,), (1,)))
    Q = _bf16_dot(P, A2, ((1,), (0,)))
    u4 = _bf16_dot(u1, Q, ((1,), (1,)))
    u4t = _bf16_dot(Q, r3_ref[0, 0, h], ((1,), (1,)))[:, 0:1]

    col_i = lax.broadcasted_iota(jnp.int32, (S, 1), 0)
    row_j = lax.broadcasted_iota(jnp.int32, (1, S), 1)
    s_row = jnp.where(row_j == 0, NEG_INF, u4)
    s_col = jnp.where(col_i == 0, NEG_INF, u4t)
    one = jnp.float32(1.0)
    zero = jnp.float32(0.0)
    g = jnp.where(s_row > s_col, one, zero)
    e = jnp.where(s_row == s_col, one, zero)
    lo = jnp.where(row_j < col_i, one, zero)
    C = (g + e * lo).astype(jnp.bfloat16)
    rank = lax.dot_general(C, jnp.ones((S, 1), jnp.bfloat16),
                           (((1,), (0,)), ((), ())),
                           preferred_element_type=jnp.float32)
    kio = lax.broadcasted_iota(jnp.int32, (1, K), 1).astype(jnp.float32)
    onehot = jnp.where(rank == kio, one, zero).astype(jnp.bfloat16)
    iv = (row_j - 1).astype(jnp.float32).astype(jnp.bfloat16)
    out_f = lax.dot_general(iv, onehot, (((1,), (0,)), ((), ())),
                            preferred_element_type=jnp.float32)
    out_ref[0, h] = out_f.astype(jnp.int32)


def _body(x0_ref, x1_ref, x2_ref, r3_ref, out_ref):
    for h in range(12):
        _one_head(x0_ref, x1_ref, x2_ref, r3_ref, out_ref, h)


@jax.jit
def kernel(x):
    L, B, H, s1, s2 = x.shape
    assert (s1, s2) == (S, S)

    grid = (B,)
    mat_spec = lambda l: pl.BlockSpec(
        (1, 1, H, S, S), lambda b, l=l: (l, b, 0, 0, 0))
    row_spec = pl.BlockSpec((1, 1, H, 8, S), lambda b: (3, b, 0, 0, 0))

    out = pl.pallas_call(
        _body,
        grid=grid,
        in_specs=[mat_spec(0), mat_spec(1), mat_spec(2), row_spec],
        out_specs=pl.BlockSpec((1, H, 1, K), lambda b: (b, 0, 0, 0)),
        out_shape=jax.ShapeDtypeStruct((B, H, 1, K), jnp.int32),
    )(x, x, x, x)
    return out.reshape(B, H, K)


# speedup vs baseline: 4.0603x; 1.0140x over previous
import jax
import jax.numpy as jnp
from jax import lax
from jax.experimental import pallas as pl

S = 197
K = 12
NEG_INF = float("-inf")


def _bf16_dot(a, b, dims):
    return lax.dot_general(a, b, (dims, ((), ())),
                           preferred_element_type=jnp.float32)


def _one_head(x0_ref, x1_ref, x2_ref, r3_ref, out_ref, h):
    A0 = x0_ref[0, 0, h]
    A1 = x1_ref[0, 0, h]
    A2 = x2_ref[0, 0, h]
    u1 = r3_ref[0, 0, h, 0:1]

    P = _bf16_dot(A0, A1, ((1,), (1,)))
    Q = _bf16_dot(P, A2, ((1,), (0,)))
    u4 = _bf16_dot(u1, Q, ((1,), (1,)))
    u4t = _bf16_dot(Q, r3_ref[0, 0, h], ((1,), (1,)))[:, 0:1]

    col_i = lax.broadcasted_iota(jnp.int32, (S, 1), 0)
    row_j = lax.broadcasted_iota(jnp.int32, (1, S), 1)
    s_row = jnp.where(row_j == 0, NEG_INF, u4)
    s_col = jnp.where(col_i == 0, NEG_INF, u4t)
    one = jnp.float32(1.0)
    zero = jnp.float32(0.0)
    g = jnp.where(s_row > s_col, one, zero)
    e = jnp.where(s_row == s_col, one, zero)
    lo = jnp.where(row_j < col_i, one, zero)
    C = g + e * lo
    rank = _bf16_dot(C, jnp.ones((S, 1), jnp.float32), ((1,), (0,)))
    kio = lax.broadcasted_iota(jnp.int32, (1, K), 1).astype(jnp.float32)
    onehot = jnp.where(rank == kio, one, zero)
    iv = (row_j - 1).astype(jnp.float32)
    out_f = _bf16_dot(iv, onehot, ((1,), (0,)))
    out_ref[0, h] = out_f.astype(jnp.int32)


def _body(x0_ref, x1_ref, x2_ref, r3_ref, out_ref):
    for h in range(12):
        _one_head(x0_ref, x1_ref, x2_ref, r3_ref, out_ref, h)


@jax.jit
def kernel(x):
    L, B, H, s1, s2 = x.shape
    assert (s1, s2) == (S, S)

    grid = (B,)
    mat_spec = lambda l: pl.BlockSpec(
        (1, 1, H, S, S), lambda b, l=l: (l, b, 0, 0, 0))
    row_spec = pl.BlockSpec((1, 1, H, 8, S), lambda b: (3, b, 0, 0, 0))

    out = pl.pallas_call(
        _body,
        grid=grid,
        in_specs=[mat_spec(0), mat_spec(1), mat_spec(2), row_spec],
        out_specs=pl.BlockSpec((1, H, 1, K), lambda b: (b, 0, 0, 0)),
        out_shape=jax.ShapeDtypeStruct((B, H, 1, K), jnp.int32),
    )(x, x, x, x)
    return out.reshape(B, H, K)


# speedup vs baseline: 5.1278x; 1.2629x over previous
import jax
import jax.numpy as jnp
from jax import lax
from jax.experimental import pallas as pl

S = 197
K = 12
H = 12
NEG_INF = float("-inf")


def _dot(a, b, dims):
    return lax.dot_general(a, b, (dims, ((), ())),
                           preferred_element_type=jnp.float32)


def _body(x0_ref, x1_ref, x2_ref, r3_ref, out_ref):
    P = [_dot(x0_ref[0, 0, h], x1_ref[0, 0, h], ((1,), (1,)))
         for h in range(H)]
    Q = [_dot(P[h], x2_ref[0, 0, h], ((1,), (0,)))
         for h in range(H)]
    u4 = [_dot(r3_ref[0, 0, h, 0:1], Q[h], ((1,), (1,))) for h in range(H)]
    u4t = [lax.transpose(u4[h], (1, 0)) for h in range(H)]

    col_i = lax.broadcasted_iota(jnp.int32, (S, 1), 0)
    row_j = lax.broadcasted_iota(jnp.int32, (1, S), 1)
    one = jnp.float32(1.0)
    zero = jnp.float32(0.0)
    lo = jnp.where(row_j < col_i, one, zero)
    kio = lax.broadcasted_iota(jnp.int32, (1, K), 1).astype(jnp.float32)
    iv = (row_j - 1).astype(jnp.float32)
    ones_col = jnp.ones((S, 1), jnp.float32)

    for h in range(H):
        s_row = jnp.where(row_j == 0, NEG_INF, u4[h])
        s_col = jnp.where(col_i == 0, NEG_INF, u4t[h])
        C = jnp.where(s_row > s_col, one,
                      jnp.where(s_row == s_col, lo, zero))
        rank = _dot(C, ones_col, ((1,), (0,)))
        onehot = jnp.where(rank == kio, one, zero)
        out_f = _dot(iv, onehot, ((1,), (0,)))
        out_ref[0, h] = out_f.astype(jnp.int32)


@jax.jit
def kernel(x):
    L, B, nh, s1, s2 = x.shape
    assert (nh, s1, s2) == (H, S, S)

    grid = (B,)
    mat_spec = lambda l: pl.BlockSpec(
        (1, 1, H, S, S), lambda b, l=l: (l, b, 0, 0, 0))
    row_spec = pl.BlockSpec((1, 1, H, 8, S), lambda b: (3, b, 0, 0, 0))

    out = pl.pallas_call(
        _body,
        grid=grid,
        in_specs=[mat_spec(0), mat_spec(1), mat_spec(2), row_spec],
        out_specs=pl.BlockSpec((1, H, 1, K), lambda b: (b, 0, 0, 0)),
        out_shape=jax.ShapeDtypeStruct((B, H, 1, K), jnp.int32),
    )(x, x, x, x)
    return out.reshape(B, H, K)


# speedup vs baseline: 5.2933x; 1.0323x over previous
import jax
import jax.numpy as jnp
from jax import lax
from jax.experimental import pallas as pl

S = 197
K = 12
H = 12
NEG_INF = float("-inf")


def _dot(a, b, dims):
    return lax.dot_general(a, b, (dims, ((), ())),
                           preferred_element_type=jnp.float32)


def _body(x0_ref, x1_ref, x2_ref, r3_ref, out_ref):
    P = [_dot(x0_ref[0, 0, h], x1_ref[0, 0, h], ((1,), (1,)))
         for h in range(H)]
    Q = [_dot(P[h], x2_ref[0, 0, h], ((1,), (0,)))
         for h in range(H)]
    u4 = [_dot(r3_ref[0, 0, h, 0:1], Q[h], ((1,), (1,))) for h in range(H)]
    u4t = [_dot(Q[h], r3_ref[0, 0, h], ((1,), (1,)))[:, 0:1] for h in range(H)]

    col_i = lax.broadcasted_iota(jnp.int32, (S, 1), 0)
    row_j = lax.broadcasted_iota(jnp.int32, (1, S), 1)
    one = jnp.float32(1.0)
    zero = jnp.float32(0.0)
    lo = jnp.where(row_j < col_i, one, zero)
    kio = lax.broadcasted_iota(jnp.int32, (1, K), 1).astype(jnp.float32)
    iv = (row_j - 1).astype(jnp.float32)
    ones_col = jnp.ones((S, 1), jnp.float32)

    for h in range(H):
        s_row = jnp.where(row_j == 0, NEG_INF, u4[h])
        s_col = jnp.where(col_i == 0, NEG_INF, u4t[h])
        C = jnp.where(s_row > s_col, one,
                      jnp.where(s_row == s_col, lo, zero))
        rank = _dot(C, ones_col, ((1,), (0,)))
        onehot = jnp.where(rank == kio, one, zero)
        out_f = _dot(iv, onehot, ((1,), (0,)))
        out_ref[0, h] = out_f.astype(jnp.int32)


@jax.jit
def kernel(x):
    L, B, nh, s1, s2 = x.shape
    assert (nh, s1, s2) == (H, S, S)

    grid = (B,)
    mat_spec = lambda l: pl.BlockSpec(
        (1, 1, H, S, S), lambda b, l=l: (l, b, 0, 0, 0))
    row_spec = pl.BlockSpec((1, 1, H, 8, S), lambda b: (3, b, 0, 0, 0))

    out = pl.pallas_call(
        _body,
        grid=grid,
        in_specs=[mat_spec(0), mat_spec(1), mat_spec(2), row_spec],
        out_specs=pl.BlockSpec((1, H, 1, K), lambda b: (b, 0, 0, 0)),
        out_shape=jax.ShapeDtypeStruct((B, H, 1, K), jnp.int32),
    )(x, x, x, x)
    return out.reshape(B, H, K)


# speedup vs baseline: 6.5315x; 1.2339x over previous
import jax
import jax.numpy as jnp
from jax import lax
from jax.experimental import pallas as pl

S = 197
K = 12
H = 12
NEG_INF = float("-inf")


def _dot(a, b, dims):
    return lax.dot_general(a, b, (dims, ((), ())),
                           preferred_element_type=jnp.float32)


def _body(x0_ref, x1_ref, x2_ref, r3_ref, out_ref):
    P = [_dot(x0_ref[0, 0, h], x1_ref[0, 0, h], ((1,), (1,)))
         for h in range(H)]
    Q = [_dot(P[h], x2_ref[0, 0, h], ((1,), (0,)))
         for h in range(H)]
    u4 = [_dot(r3_ref[0, 0, h, 0:1], Q[h], ((1,), (1,))) for h in range(H)]
    u4t = [_dot(Q[h], r3_ref[0, 0, h], ((1,), (1,)))[:, 0:1] for h in range(H)]

    col_i = lax.broadcasted_iota(jnp.int32, (S, 1), 0)
    row_j = lax.broadcasted_iota(jnp.int32, (1, S), 1)
    one = jnp.float32(1.0)
    zero = jnp.float32(0.0)
    lo = jnp.where(row_j < col_i, one, zero)
    kio = lax.broadcasted_iota(jnp.int32, (1, K), 1).astype(jnp.float32)
    ivc = (col_i - 1).astype(jnp.float32)
    ones_col = jnp.ones((S, 1), jnp.float32)

    Cs = []
    for h in range(H):
        s_row = jnp.where(row_j == 0, NEG_INF, u4[h])
        s_col = jnp.where(col_i == 0, NEG_INF, u4t[h])
        Cs.append(jnp.where(s_row > s_col, one,
                            jnp.where(s_row == s_col, lo, zero)))
    ranks = [_dot(Cs[h], ones_col, ((1,), (0,))) for h in range(H)]
    for h in range(H):
        onehot = jnp.where(ranks[h] == kio, one, zero)
        out_f = jnp.sum(onehot * ivc, axis=0, keepdims=True)
        out_ref[0, h] = out_f.astype(jnp.int32)


@jax.jit
def kernel(x):
    L, B, nh, s1, s2 = x.shape
    assert (nh, s1, s2) == (H, S, S)

    grid = (B,)
    mat_spec = lambda l: pl.BlockSpec(
        (1, 1, H, S, S), lambda b, l=l: (l, b, 0, 0, 0))
    row_spec = pl.BlockSpec((1, 1, H, 8, S), lambda b: (3, b, 0, 0, 0))

    out = pl.pallas_call(
        _body,
        grid=grid,
        in_specs=[mat_spec(0), mat_spec(1), mat_spec(2), row_spec],
        out_specs=pl.BlockSpec((1, H, 1, K), lambda b: (b, 0, 0, 0)),
        out_shape=jax.ShapeDtypeStruct((B, H, 1, K), jnp.int32),
    )(x, x, x, x)
    return out.reshape(B, H, K)
